# baseline (speedup 1.0000x reference)
import functools

import jax
import jax.numpy as jnp
from jax import lax
from jax.experimental import pallas as pl
from jax.experimental.pallas import tpu as pltpu
from jax.experimental.pallas import tpu_sc as plsc

B = 16
L = 4096
D = 1024

NC = 2
NS = 16
LANES = 16

CHUNK = 64
DH = D // 2
NV = DH // LANES


def _sc_body(padded_hbm, lens_hbm, out_hbm, lens_v, buf, obuf):
    c = lax.axis_index("c")
    s = lax.axis_index("s")
    wid = s * NC + c
    b = wid // 2
    h = wid % 2

    pltpu.sync_copy(lens_hbm, lens_v.at[pl.ds(0, B)])
    n = lens_v[pl.ds(b, LANES)][0]

    init_scalar = lax.select(n > 0, -jnp.inf, 0.0)
    init = lax.broadcast_in_dim(init_scalar, (LANES,), ())
    accs = tuple(init for _ in range(NV))

    nch = (n + CHUNK - 1) // CHUNK

    def chunk_body(ci, accs):
        pltpu.sync_copy(
            padded_hbm.at[b, pl.ds(ci * CHUNK, CHUNK), pl.ds(h * DH, DH)], buf
        )
        rows = jnp.minimum(n - ci * CHUNK, CHUNK)

        def row_body(r, accs):
            return tuple(
                jnp.maximum(accs[v], buf[r, pl.ds(v * LANES, LANES)])
                for v in range(NV)
            )

        return lax.fori_loop(0, rows, row_body, accs)

    accs = lax.fori_loop(0, nch, chunk_body, accs)

    for v in range(NV):
        obuf[pl.ds(v * LANES, LANES)] = accs[v]
    pltpu.sync_copy(obuf, out_hbm.at[b, pl.ds(h * DH, DH)])


@functools.partial(jax.jit, static_argnums=())
def _pooled(padded, lens):
    mesh = plsc.VectorSubcoreMesh(
        core_axis_name="c", subcore_axis_name="s", num_cores=NC, num_subcores=NS
    )
    k = pl.kernel(
        _sc_body,
        out_type=jax.ShapeDtypeStruct((B, D), jnp.float32),
        mesh=mesh,
        scratch_types=[
            pltpu.VMEM((2 * B,), jnp.int32),
            pltpu.VMEM((CHUNK, DH), jnp.float32),
            pltpu.VMEM((DH,), jnp.float32),
        ],
    )
    return k(padded, lens)


def kernel(sentence_embeddings_list, num_sentences, encoder_hidden_size):
    del encoder_hidden_size
    padded = sentence_embeddings_list.astype(jnp.float32)
    lens = num_sentences.astype(jnp.int32)
    return _pooled(padded, lens)


# speedup vs baseline: 2.2343x; 2.2343x over previous
import functools

import jax
import jax.numpy as jnp
from jax import lax
from jax.experimental import pallas as pl
from jax.experimental.pallas import tpu as pltpu
from jax.experimental.pallas import tpu_sc as plsc

B = 16
L = 4096
D = 1024

NC = 2
NS = 16
LANES = 16

BPC = B // NC
NSEG = 2 * BPC
CHUNK = 64
DH = D // 2
NV = DH // LANES

_NEG = float("-inf")


def _sc_body(padded_hbm, lens_hbm, out_hbm,
             lens_v, buf0, buf1, obuf, cbuf, partials, sem0, sem1):
    c = lax.axis_index("c")
    s = lax.axis_index("s")

    pltpu.sync_copy(lens_hbm, lens_v.at[pl.ds(0, B)])

    def seg_len(j):
        return lens_v[pl.ds(BPC * c + j // 2, LANES)][0]

    neg = jnp.full((LANES,), _NEG, jnp.float32)

    for v in range(NV):
        obuf[pl.ds(v * LANES, LANES)] = neg

    def init_body(j, carry):
        pltpu.sync_copy(obuf, partials.at[j, s])
        return carry

    lax.fori_loop(0, NSEG, init_body, 0)

    def sum_body(j, tot):
        return tot + seg_len(j)

    total = lax.fori_loop(0, NSEG, sum_body, jnp.int32(0))
    u0 = (s * total) // NS
    u1 = ((s + 1) * total) // NS

    def seg_body(j, start):
        n = seg_len(j)
        b = BPC * c + j // 2
        h = j % 2
        lo = jnp.clip(u0 - start, 0, n)
        hi = jnp.clip(u1 - start, 0, n)

        @pl.when(lo < hi)
        def _process():
            c0 = lo // CHUNK
            nch = (hi + CHUNK - 1) // CHUNK - c0

            def src(ci):
                return padded_hbm.at[
                    b, pl.ds((c0 + ci) * CHUNK, CHUNK), pl.ds(h * DH, DH)
                ]

            pltpu.async_copy(src(0), buf0, sem0)

            def rows(ci, buf, accs):
                r0 = jnp.clip(lo - (c0 + ci) * CHUNK, 0, CHUNK)
                r1 = jnp.clip(hi - (c0 + ci) * CHUNK, 0, CHUNK)

                def row_body(r, accs):
                    return tuple(
                        jnp.maximum(accs[v], buf[r, pl.ds(v * LANES, LANES)])
                        for v in range(NV)
                    )

                return lax.fori_loop(r0, r1, row_body, accs)

            def pair_body(k, accs):
                ca = 2 * k
                cb = 2 * k + 1

                @pl.when(cb < nch)
                def _():
                    pltpu.async_copy(src(cb), buf1, sem1)

                pltpu.make_async_copy(src(ca), buf0, sem0).wait()
                accs = rows(ca, buf0, accs)

                @pl.when(ca + 2 < nch)
                def _():
                    pltpu.async_copy(src(ca + 2), buf0, sem0)

                @pl.when(cb < nch)
                def _():
                    pltpu.make_async_copy(src(cb), buf1, sem1).wait()

                accs = rows(cb, buf1, accs)
                return accs

            npairs = (nch + 1) // 2
            accs = lax.fori_loop(
                0, npairs, pair_body, tuple(neg for _ in range(NV))
            )

            for v in range(NV):
                obuf[pl.ds(v * LANES, LANES)] = accs[v]
            pltpu.sync_copy(obuf, partials.at[j, s])

        return start + n

    lax.fori_loop(0, NSEG, seg_body, jnp.int32(0))

    plsc.subcore_barrier()

    n_s = seg_len(s)
    b_s = BPC * c + s // 2
    h_s = s % 2
    pltpu.sync_copy(partials.at[s], cbuf)

    def comb_body(w, accs):
        return tuple(
            jnp.maximum(accs[v], cbuf[w, pl.ds(v * LANES, LANES)])
            for v in range(NV)
        )

    accs = lax.fori_loop(0, NS, comb_body, tuple(neg for _ in range(NV)))
    for v in range(NV):
        obuf[pl.ds(v * LANES, LANES)] = accs[v]

    @pl.when(n_s == 0)
    def _zeros():
        z = jnp.zeros((LANES,), jnp.float32)
        for v in range(NV):
            obuf[pl.ds(v * LANES, LANES)] = z

    pltpu.sync_copy(obuf, out_hbm.at[b_s, pl.ds(h_s * DH, DH)])


@jax.jit
def _pooled(padded, lens):
    mesh = plsc.VectorSubcoreMesh(
        core_axis_name="c", subcore_axis_name="s", num_cores=NC, num_subcores=NS
    )
    k = pl.kernel(
        _sc_body,
        out_type=jax.ShapeDtypeStruct((B, D), jnp.float32),
        mesh=mesh,
        scratch_types=[
            pltpu.VMEM((2 * B,), jnp.int32),
            pltpu.VMEM((CHUNK, DH), jnp.float32),
            pltpu.VMEM((CHUNK, DH), jnp.float32),
            pltpu.VMEM((DH,), jnp.float32),
            pltpu.VMEM((NS, DH), jnp.float32),
            pltpu.VMEM_SHARED((NSEG, NS, DH), jnp.float32),
            pltpu.SemaphoreType.DMA,
            pltpu.SemaphoreType.DMA,
        ],
    )
    return k(padded, lens)


def kernel(sentence_embeddings_list, num_sentences, encoder_hidden_size):
    del encoder_hidden_size
    padded = sentence_embeddings_list.astype(jnp.float32)
    lens = num_sentences.astype(jnp.int32)
    return _pooled(padded, lens)
